# baseline (speedup 1.0000x reference)
import functools

import jax
import jax.numpy as jnp
from jax import lax
from jax.experimental import pallas as pl
from jax.experimental.pallas import tpu as pltpu
from jax.experimental.pallas import tpu_sc as plsc

_HARTREE_TO_EV = 27.211386245988
_CUTOFF = 5.0
_SPSHIFT = 20

_info = plsc.get_sparse_core_info()
_NC, _NS, _L = _info.num_cores, _info.num_subcores, _info.num_lanes
_NW = _NC * _NS


def _rsqrt_newton(x):
    xi = plsc.bitcast(x, jnp.int32)
    y = plsc.bitcast(jnp.int32(0x5F3759DF) - (xi >> 1), jnp.float32)
    for _ in range(3):
        y = y * (1.5 - 0.5 * x * y * y)
    return y


def _soft_exp(x):
    t = x * 1.4426950408889634
    tt = t + 0.5
    ti = tt.astype(jnp.int32)
    n = ti - jnp.where(tt < ti.astype(jnp.float32), 1, 0)
    u = (t - n.astype(jnp.float32)) * 0.6931471805599453
    p = 1.0 / 720.0
    for c in (1.0 / 120.0, 1.0 / 24.0, 1.0 / 6.0, 0.5, 1.0, 1.0):
        p = p * u + c
    scale = plsc.bitcast((jnp.maximum(n, -127) + 127) << 23, jnp.float32)
    return scale * p


def _make_sc_kernel(npa, ppw, npass):
    quarter = npa // npass
    mask_lo = (1 << _SPSHIFT) - 1

    def body(cx_h, cy_h, cz_h, pa_h, pb_h, gsum_out, s_out,
             cx_v, cy_v, cz_v, pa_v, pb_v, gsum_v, ge_v, s_v):
        wid = lax.axis_index("s") * _NC + lax.axis_index("c")
        base = wid * ppw
        pltpu.sync_copy(cx_h, cx_v)
        pltpu.sync_copy(cy_h, cy_v)
        pltpu.sync_copy(cz_h, cz_v)
        pltpu.sync_copy(pa_h.at[pl.ds(base, ppw)], pa_v)
        pltpu.sync_copy(pb_h.at[pl.ds(base, ppw)], pb_v)

        zeros = jnp.zeros((_L,), jnp.float32)

        @plsc.parallel_loop(0, npa, _L, unroll=8)
        def zero_gsum(i):
            gsum_v[pl.ds(i, _L)] = zeros

        def zero_s():
            for z in range(10):
                @plsc.parallel_loop(0, quarter, _L, unroll=8)
                def zero_row(i):
                    s_v[z, pl.ds(i, _L)] = zeros

        zero_s()

        @plsc.parallel_loop(0, ppw, _L, unroll=4)
        def pass0(i):
            pa16 = pa_v[pl.ds(i, _L)]
            pb16 = pb_v[pl.ds(i, _L)]
            a16 = pa16 & mask_lo
            b16 = pb16 & mask_lo
            spb = pa16 >> _SPSHIFT
            spa = pb16 >> _SPSHIFT
            xa = plsc.load_gather(cx_v, [a16])
            ya = plsc.load_gather(cy_v, [a16])
            za = plsc.load_gather(cz_v, [a16])
            xb = plsc.load_gather(cx_v, [b16])
            yb = plsc.load_gather(cy_v, [b16])
            zb = plsc.load_gather(cz_v, [b16])
            dx = xa - xb
            dy = ya - yb
            dz = za - zb
            d2 = dx * dx + dy * dy + dz * dz + 1e-6
            d = d2 * _rsqrt_newton(d2)
            glr = _soft_exp(-0.1 * d)
            g2 = glr * glr
            g5 = g2 * g2 * glr
            ge = jnp.where(d <= _CUTOFF, g5 * g5, 0.0)
            ge_v[pl.ds(i, _L)] = ge
            plsc.addupdate_scatter(gsum_v, [a16], glr)
            plsc.addupdate_scatter(gsum_v, [b16], glr)
            ma = a16 < quarter
            mb = b16 < quarter
            plsc.addupdate_scatter(s_v, [spb, jnp.where(ma, a16, 0)], ge,
                                   mask=ma)
            plsc.addupdate_scatter(s_v, [spa, jnp.where(mb, b16, 0)], ge,
                                   mask=mb)

        pltpu.sync_copy(gsum_v, gsum_out.at[wid])
        pltpu.sync_copy(s_v, s_out.at[wid, 0])

        for pno in range(1, npass):
            zero_s()
            lo = pno * quarter

            @plsc.parallel_loop(0, ppw, _L, unroll=4)
            def passk(i):
                pa16 = pa_v[pl.ds(i, _L)]
                pb16 = pb_v[pl.ds(i, _L)]
                ge = ge_v[pl.ds(i, _L)]
                al = (pa16 & mask_lo) - lo
                bl = (pb16 & mask_lo) - lo
                spb = pa16 >> _SPSHIFT
                spa = pb16 >> _SPSHIFT
                ma = (al >= 0) & (al < quarter)
                mb = (bl >= 0) & (bl < quarter)
                plsc.addupdate_scatter(s_v, [spb, jnp.where(ma, al, 0)], ge,
                                       mask=ma)
                plsc.addupdate_scatter(s_v, [spa, jnp.where(mb, bl, 0)], ge,
                                       mask=mb)

            pltpu.sync_copy(s_v, s_out.at[wid, pno])

    return pl.kernel(
        body,
        out_type=[
            jax.ShapeDtypeStruct((_NW, npa), jnp.float32),
            jax.ShapeDtypeStruct((_NW, npass, 10, quarter), jnp.float32),
        ],
        mesh=plsc.VectorSubcoreMesh(core_axis_name="c", subcore_axis_name="s"),
        compiler_params=pltpu.CompilerParams(needs_layout_passes=False),
        scratch_types=[
            pltpu.VMEM((npa,), jnp.float32),
            pltpu.VMEM((npa,), jnp.float32),
            pltpu.VMEM((npa,), jnp.float32),
            pltpu.VMEM((ppw,), jnp.int32),
            pltpu.VMEM((ppw,), jnp.int32),
            pltpu.VMEM((npa,), jnp.float32),
            pltpu.VMEM((ppw,), jnp.float32),
            pltpu.VMEM((10, quarter), jnp.float32),
        ],
    )


def _tc_tail(n_species, s_ref, gs_ref, sp_ref, embt_ref, w1t_ref, w2t_ref,
             e_ref, q_ref):
    j = pl.program_id(0)
    hi = lax.Precision.HIGHEST
    s_sum = s_ref[0, 0]
    g_sum = gs_ref[0:1, :]
    for w in range(1, _NW):
        s_sum = s_sum + s_ref[w, 0]
        g_sum = g_sum + gs_ref[w:w + 1, :]
    s16 = jnp.concatenate(
        [s_sum, jnp.zeros((16 - n_species, 128), jnp.float32)], axis=0)
    embt = embt_ref[...]
    msgt = jnp.dot(embt, s16, precision=hi,
                   preferred_element_type=jnp.float32)
    row = lax.broadcasted_iota(jnp.int32, (16, 128), 0)
    oh = (row == sp_ref[0]).astype(jnp.float32)
    featt = jnp.dot(embt, oh, precision=hi,
                    preferred_element_type=jnp.float32)
    pt = jnp.dot(w1t_ref[...], msgt,
                 preferred_element_type=jnp.float32)
    e_blk = jnp.sum(featt * pt) / _HARTREE_TO_EV

    @pl.when(j == 0)
    def _():
        e_ref[0, 0] = 0.0

    e_ref[0, 0] += e_blk

    cqt = jnp.dot(w2t_ref[...], featt,
                  preferred_element_type=jnp.float32)
    q_ref[0] = cqt[0:1, :] * jnp.tanh(g_sum)


def kernel(species, coords, pair_idx, emb, w1, w2):
    n = coords.shape[1]
    p = pair_idx.shape[1]
    z, dim = emb.shape
    f32 = jnp.float32
    npass = 4

    npa = ((n + 2 + 511) // 512) * 512
    quarter = npa // npass
    nrow = npa // 128
    ppw = ((p + _NW * _L - 1) // (_NW * _L)) * _L
    pp = ppw * _NW

    c = coords[0]
    cx = jnp.zeros((npa,), f32).at[:n].set(c[:, 0]).at[n + 1].set(1e4)
    cy = jnp.zeros((npa,), f32).at[:n].set(c[:, 1])
    cz = jnp.zeros((npa,), f32).at[:n].set(c[:, 2])
    sp = jnp.zeros((npa,), jnp.int32).at[:n].set(species[0].astype(jnp.int32))
    pad_a = jnp.full((pp - p,), n, jnp.int32)
    pad_b = jnp.full((pp - p,), n + 1, jnp.int32)
    a_idx = jnp.concatenate([pair_idx[0].astype(jnp.int32), pad_a])
    b_idx = jnp.concatenate([pair_idx[1].astype(jnp.int32), pad_b])
    pa = a_idx | (sp[b_idx] << _SPSHIFT)
    pb = b_idx | (sp[a_idx] << _SPSHIFT)

    gsums, s_part = _make_sc_kernel(npa, ppw, npass)(cx, cy, cz, pa, pb)

    embt = jnp.zeros((128, 16), f32).at[:dim, :z].set(emb.T)
    w1t = jnp.zeros((128, 128), f32).at[:dim, :dim].set(w1.T)
    w2t = jnp.zeros((8, 128), f32).at[0, :dim].set(w2[:, 0])

    bpq = quarter // 128
    energy, charges = pl.pallas_call(
        functools.partial(_tc_tail, z),
        grid=(nrow,),
        in_specs=[
            pl.BlockSpec((_NW, 1, 10, 128),
                         lambda j: (0, j // bpq, 0, j % bpq)),
            pl.BlockSpec((_NW, 128), lambda j: (0, j)),
            pl.BlockSpec((1, 1, 128), lambda j: (j, 0, 0)),
            pl.BlockSpec((128, 16), lambda j: (0, 0)),
            pl.BlockSpec((128, 128), lambda j: (0, 0)),
            pl.BlockSpec((8, 128), lambda j: (0, 0)),
        ],
        out_specs=[
            pl.BlockSpec((1, 1), lambda j: (0, 0),
                         memory_space=pltpu.MemorySpace.SMEM),
            pl.BlockSpec((1, 1, 128), lambda j: (j, 0, 0)),
        ],
        out_shape=[
            jax.ShapeDtypeStruct((1, 1), f32),
            jax.ShapeDtypeStruct((nrow, 1, 128), f32),
        ],
    )(
        s_part,
        gsums,
        sp.reshape(nrow, 1, 128),
        embt,
        w1t,
        w2t,
    )
    return energy.reshape(1), charges.reshape(-1)[:n]


# speedup vs baseline: 13.5117x; 13.5117x over previous
import functools

import jax
import jax.numpy as jnp
from jax import lax
from jax.experimental import pallas as pl
from jax.experimental.pallas import tpu as pltpu
from jax.experimental.pallas import tpu_sc as plsc

_HARTREE_TO_EV = 27.211386245988
_CUTOFF = 5.0
_SPSHIFT = 20

_info = plsc.get_sparse_core_info()
_NC, _NS, _L = _info.num_cores, _info.num_subcores, _info.num_lanes
_NW = _NC * _NS


def _rsqrt_newton(x):
    xi = plsc.bitcast(x, jnp.int32)
    y = plsc.bitcast(jnp.int32(0x5F3759DF) - (xi >> 1), jnp.float32)
    for _ in range(3):
        y = y * (1.5 - 0.5 * x * y * y)
    return y


def _soft_exp(x):
    t = x * 1.4426950408889634
    tt = t + 0.5
    ti = tt.astype(jnp.int32)
    n = ti - jnp.where(tt < ti.astype(jnp.float32), 1, 0)
    u = (t - n.astype(jnp.float32)) * 0.6931471805599453
    p = 1.0 / 720.0
    for c in (1.0 / 120.0, 1.0 / 24.0, 1.0 / 6.0, 0.5, 1.0, 1.0):
        p = p * u + c
    scale = plsc.bitcast((jnp.maximum(n, -127) + 127) << 23, jnp.float32)
    return scale * p


def _make_sc_kernel(npa, ppw, npass):
    quarter = npa // npass
    mask_lo = (1 << _SPSHIFT) - 1

    def body(cx_h, cy_h, cz_h, sp_h, pa_h, pb_h, gsum_out, s_out,
             cx_v, cy_v, cz_v, sp_v, pa_v, pb_v, gsum_v, ge_v, s_v):
        wid = lax.axis_index("s") * _NC + lax.axis_index("c")
        base = wid * ppw
        pltpu.sync_copy(cx_h, cx_v)
        pltpu.sync_copy(cy_h, cy_v)
        pltpu.sync_copy(cz_h, cz_v)
        pltpu.sync_copy(sp_h, sp_v)
        pltpu.sync_copy(pa_h.at[pl.ds(base, ppw)], pa_v)
        pltpu.sync_copy(pb_h.at[pl.ds(base, ppw)], pb_v)

        zeros = jnp.zeros((_L,), jnp.float32)

        @plsc.parallel_loop(0, npa, _L, unroll=8)
        def zero_gsum(i):
            gsum_v[pl.ds(i, _L)] = zeros

        def zero_s():
            for z in range(10):
                @plsc.parallel_loop(0, quarter, _L, unroll=8)
                def zero_row(i):
                    s_v[z, pl.ds(i, _L)] = zeros

        zero_s()

        @plsc.parallel_loop(0, ppw, _L, unroll=4)
        def pass0(i):
            a16 = pa_v[pl.ds(i, _L)]
            b16 = pb_v[pl.ds(i, _L)]
            spa = plsc.load_gather(sp_v, [a16])
            spb = plsc.load_gather(sp_v, [b16])
            pa_v[pl.ds(i, _L)] = a16 | (spb << _SPSHIFT)
            pb_v[pl.ds(i, _L)] = b16 | (spa << _SPSHIFT)
            xa = plsc.load_gather(cx_v, [a16])
            ya = plsc.load_gather(cy_v, [a16])
            za = plsc.load_gather(cz_v, [a16])
            xb = plsc.load_gather(cx_v, [b16])
            yb = plsc.load_gather(cy_v, [b16])
            zb = plsc.load_gather(cz_v, [b16])
            dx = xa - xb
            dy = ya - yb
            dz = za - zb
            d2 = dx * dx + dy * dy + dz * dz + 1e-6
            d = d2 * _rsqrt_newton(d2)
            glr = _soft_exp(-0.1 * d)
            g2 = glr * glr
            g5 = g2 * g2 * glr
            ge = jnp.where(d <= _CUTOFF, g5 * g5, 0.0)
            ge_v[pl.ds(i, _L)] = ge
            plsc.addupdate_scatter(gsum_v, [a16], glr)
            plsc.addupdate_scatter(gsum_v, [b16], glr)
            ma = a16 < quarter
            mb = b16 < quarter
            plsc.addupdate_scatter(s_v, [spb, jnp.where(ma, a16, 0)], ge,
                                   mask=ma)
            plsc.addupdate_scatter(s_v, [spa, jnp.where(mb, b16, 0)], ge,
                                   mask=mb)

        pltpu.sync_copy(gsum_v, gsum_out.at[wid])
        pltpu.sync_copy(s_v, s_out.at[wid, 0])

        for pno in range(1, npass):
            zero_s()
            lo = pno * quarter

            @plsc.parallel_loop(0, ppw, _L, unroll=4)
            def passk(i):
                pa16 = pa_v[pl.ds(i, _L)]
                pb16 = pb_v[pl.ds(i, _L)]
                ge = ge_v[pl.ds(i, _L)]
                al = (pa16 & mask_lo) - lo
                bl = (pb16 & mask_lo) - lo
                spb = pa16 >> _SPSHIFT
                spa = pb16 >> _SPSHIFT
                ma = (al >= 0) & (al < quarter)
                mb = (bl >= 0) & (bl < quarter)
                plsc.addupdate_scatter(s_v, [spb, jnp.where(ma, al, 0)], ge,
                                       mask=ma)
                plsc.addupdate_scatter(s_v, [spa, jnp.where(mb, bl, 0)], ge,
                                       mask=mb)

            pltpu.sync_copy(s_v, s_out.at[wid, pno])

    return pl.kernel(
        body,
        out_type=[
            jax.ShapeDtypeStruct((_NW, npa), jnp.float32),
            jax.ShapeDtypeStruct((_NW, npass, 10, quarter), jnp.float32),
        ],
        mesh=plsc.VectorSubcoreMesh(core_axis_name="c", subcore_axis_name="s"),
        compiler_params=pltpu.CompilerParams(needs_layout_passes=False),
        scratch_types=[
            pltpu.VMEM((npa,), jnp.float32),
            pltpu.VMEM((npa,), jnp.float32),
            pltpu.VMEM((npa,), jnp.float32),
            pltpu.VMEM((npa,), jnp.int32),
            pltpu.VMEM((ppw,), jnp.int32),
            pltpu.VMEM((ppw,), jnp.int32),
            pltpu.VMEM((npa,), jnp.float32),
            pltpu.VMEM((ppw,), jnp.float32),
            pltpu.VMEM((10, quarter), jnp.float32),
        ],
    )


def _tc_tail(n_species, s_ref, gs_ref, sp_ref, embt_ref, w1t_ref, w2t_ref,
             e_ref, q_ref):
    j = pl.program_id(0)
    hi = lax.Precision.HIGHEST
    s_sum = s_ref[0, 0]
    g_sum = gs_ref[0:1, :]
    for w in range(1, _NW):
        s_sum = s_sum + s_ref[w, 0]
        g_sum = g_sum + gs_ref[w:w + 1, :]
    s16 = jnp.concatenate(
        [s_sum, jnp.zeros((16 - n_species, 128), jnp.float32)], axis=0)
    embt = embt_ref[...]
    msgt = jnp.dot(embt, s16, precision=hi,
                   preferred_element_type=jnp.float32)
    row = lax.broadcasted_iota(jnp.int32, (16, 128), 0)
    oh = (row == sp_ref[0]).astype(jnp.float32)
    featt = jnp.dot(embt, oh, precision=hi,
                    preferred_element_type=jnp.float32)
    pt = jnp.dot(w1t_ref[...], msgt,
                 preferred_element_type=jnp.float32)
    e_blk = jnp.sum(featt * pt) / _HARTREE_TO_EV

    @pl.when(j == 0)
    def _():
        e_ref[0, 0] = 0.0

    e_ref[0, 0] += e_blk

    cqt = jnp.dot(w2t_ref[...], featt,
                  preferred_element_type=jnp.float32)
    q_ref[0] = cqt[0:1, :] * jnp.tanh(g_sum)


def kernel(species, coords, pair_idx, emb, w1, w2):
    n = coords.shape[1]
    p = pair_idx.shape[1]
    z, dim = emb.shape
    f32 = jnp.float32
    npass = 4

    npa = ((n + 2 + 511) // 512) * 512
    quarter = npa // npass
    nrow = npa // 128
    ppw = ((p + _NW * _L - 1) // (_NW * _L)) * _L
    pp = ppw * _NW

    c = coords[0]
    cx = jnp.zeros((npa,), f32).at[:n].set(c[:, 0]).at[n + 1].set(1e4)
    cy = jnp.zeros((npa,), f32).at[:n].set(c[:, 1])
    cz = jnp.zeros((npa,), f32).at[:n].set(c[:, 2])
    sp = jnp.zeros((npa,), jnp.int32).at[:n].set(species[0].astype(jnp.int32))
    pad_a = jnp.full((pp - p,), n, jnp.int32)
    pad_b = jnp.full((pp - p,), n + 1, jnp.int32)
    a_idx = jnp.concatenate([pair_idx[0].astype(jnp.int32), pad_a])
    b_idx = jnp.concatenate([pair_idx[1].astype(jnp.int32), pad_b])

    gsums, s_part = _make_sc_kernel(npa, ppw, npass)(cx, cy, cz, sp,
                                                     a_idx, b_idx)

    embt = jnp.zeros((128, 16), f32).at[:dim, :z].set(emb.T)
    w1t = jnp.zeros((128, 128), f32).at[:dim, :dim].set(w1.T)
    w2t = jnp.zeros((8, 128), f32).at[0, :dim].set(w2[:, 0])

    bpq = quarter // 128
    energy, charges = pl.pallas_call(
        functools.partial(_tc_tail, z),
        grid=(nrow,),
        in_specs=[
            pl.BlockSpec((_NW, 1, 10, 128),
                         lambda j: (0, j // bpq, 0, j % bpq)),
            pl.BlockSpec((_NW, 128), lambda j: (0, j)),
            pl.BlockSpec((1, 1, 128), lambda j: (j, 0, 0)),
            pl.BlockSpec((128, 16), lambda j: (0, 0)),
            pl.BlockSpec((128, 128), lambda j: (0, 0)),
            pl.BlockSpec((8, 128), lambda j: (0, 0)),
        ],
        out_specs=[
            pl.BlockSpec((1, 1), lambda j: (0, 0),
                         memory_space=pltpu.MemorySpace.SMEM),
            pl.BlockSpec((1, 1, 128), lambda j: (j, 0, 0)),
        ],
        out_shape=[
            jax.ShapeDtypeStruct((1, 1), f32),
            jax.ShapeDtypeStruct((nrow, 1, 128), f32),
        ],
    )(
        s_part,
        gsums,
        sp.reshape(nrow, 1, 128),
        embt,
        w1t,
        w2t,
    )
    return energy.reshape(1), charges.reshape(-1)[:n]
